# baseline (speedup 1.0000x reference)
import jax
import jax.numpy as jnp
from jax import lax
from jax.experimental import pallas as pl
from jax.experimental.pallas import tpu as pltpu
from jax.experimental.pallas import tpu_sc as plsc

EMBED = 64
PADW = 128
SBLK = 128
TT = 2
NC, NS = 2, 16
NW = NC * NS
LANES = 16


def _make_lookup(n_sent: int, s_len: int):
    assert n_sent == NW * SBLK and s_len % TT == 0
    n_groups = s_len // TT
    idx_per_w = SBLK * s_len
    mesh = plsc.VectorSubcoreMesh(core_axis_name="c", subcore_axis_name="s",
                                  num_cores=NC, num_subcores=NS)

    def body(src_idx, tgt_idx, src_tab, tgt_tab, out_src, out_tgt,
             idx_v, list_v, gbuf0, gbuf1, blk0, blk1, gsem, ssem):
        wid = lax.axis_index("s") * NC + lax.axis_index("c")
        ib50 = lax.iota(jnp.int32, LANES) * s_len
        ibl = lax.iota(jnp.int32, LANES)
        gbufs = (gbuf0, gbuf1)
        blks = (blk0, blk1)

        for idx_hbm, tab, out_hbm in ((src_idx, src_tab, out_src),
                                      (tgt_idx, tgt_tab, out_tgt)):
            pltpu.sync_copy(idx_hbm.at[pl.ds(wid * idx_per_w, idx_per_w)],
                            idx_v)

            def repack_and_fire(g, gbuf, tab=tab):
                for c in range(2 * SBLK // LANES):
                    tt, sb = c // (SBLK // LANES), (c % (SBLK // LANES))
                    pos = ib50 + (sb * LANES * s_len + g * TT + tt)
                    vals = plsc.load_gather(idx_v, [pos])
                    list_v[pl.ds(c * LANES, LANES)] = vals
                for tt in range(TT):
                    pltpu.async_copy(
                        tab.at[list_v.at[pl.ds(tt * SBLK, SBLK)]],
                        gbuf.at[pl.ds(tt * SBLK, SBLK)], gsem)

            def drain_gathers(gbuf, tab=tab):
                pltpu.make_async_copy(tab.at[pl.ds(0, TT * SBLK)], gbuf,
                                      gsem).wait()

            def drain_stores(out_hbm=out_hbm):
                for blk in blks:
                    pltpu.make_async_copy(
                        blk, out_hbm.at[0, :, pl.ds(wid * SBLK, SBLK)],
                        ssem).wait()

            repack_and_fire(0, gbufs[0])

            @pl.loop(0, n_groups + n_groups % 2, step=2)
            def _(g, tab=tab, out_hbm=out_hbm):
                for b in range(2):
                    gg = g + b

                    @pl.when(gg < n_groups)
                    def _():
                        drain_gathers(gbufs[b])

                        @pl.when(gg + 1 < n_groups)
                        def _():
                            repack_and_fire(gg + 1, gbufs[1 - b])

                        @pl.when(gg > 0)
                        def _():
                            drain_stores()

                        for tt in range(TT):
                            blk = blks[tt]

                            @pl.loop(0, EMBED)
                            def _(e, tt=tt, blk=blk):
                                col = jnp.full((LANES,), e, jnp.int32)
                                for c in range(SBLK // LANES):
                                    row = ibl + (tt * SBLK + c * LANES)
                                    v = plsc.load_gather(gbufs[b], [row, col])
                                    blk[e, pl.ds(c * LANES, LANES)] = v

                            pltpu.async_copy(
                                blk,
                                out_hbm.at[gg * TT + tt, :,
                                           pl.ds(wid * SBLK, SBLK)], ssem)
            drain_stores()

    out_sd = jax.ShapeDtypeStruct((s_len, EMBED, n_sent), jnp.float32)
    return pl.kernel(
        body,
        out_type=(out_sd, out_sd),
        mesh=mesh,
        scratch_types=[
            pltpu.VMEM((SBLK * s_len,), jnp.int32),
            pltpu.VMEM((TT * SBLK,), jnp.int32),
            pltpu.VMEM((TT * SBLK, PADW), jnp.float32),
            pltpu.VMEM((TT * SBLK, PADW), jnp.float32),
            pltpu.VMEM((EMBED, SBLK), jnp.float32),
            pltpu.VMEM((EMBED, SBLK), jnp.float32),
            pltpu.SemaphoreType.DMA,
            pltpu.SemaphoreType.DMA,
        ],
        compiler_params=pltpu.CompilerParams(needs_layout_passes=False),
    )


def kernel(src_indices, tgt_indices, src_table, tgt_table):
    b, s = src_indices.shape
    src_flat = src_indices.reshape(b * s).astype(jnp.int32)
    tgt_flat = tgt_indices.reshape(b * s).astype(jnp.int32)
    src_pad = jnp.pad(src_table, ((0, 0), (0, PADW - EMBED)))
    tgt_pad = jnp.pad(tgt_table, ((0, 0), (0, PADW - EMBED)))
    p_src, p_tgt = _make_lookup(b, s)(src_flat, tgt_flat, src_pad, tgt_pad)
    return (jnp.transpose(p_src, (2, 0, 1)), jnp.transpose(p_tgt, (2, 0, 1)))


# speedup vs baseline: 1.1772x; 1.1772x over previous
import jax
import jax.numpy as jnp
from jax import lax
from jax.experimental import pallas as pl
from jax.experimental.pallas import tpu as pltpu
from jax.experimental.pallas import tpu_sc as plsc

EMBED = 64
PADW = 128
SBLK = 128
TT = 2
NC, NS = 2, 16
NW = NC * NS
LANES = 16


def _make_lookup(n_sent: int, s_len: int):
    assert n_sent == NW * SBLK and s_len % TT == 0
    n_groups = s_len // TT
    idx_per_w = SBLK * s_len
    mesh = plsc.VectorSubcoreMesh(core_axis_name="c", subcore_axis_name="s",
                                  num_cores=NC, num_subcores=NS)

    def body(src_idx, tgt_idx, src_tab, tgt_tab, out_src, out_tgt,
             idx_v, list_v, gbuf0, gbuf1, blk0, blk1, gsem, ssem):
        wid = lax.axis_index("s") * NC + lax.axis_index("c")
        ib50 = lax.iota(jnp.int32, LANES) * s_len
        iks = [lax.iota(jnp.int32, LANES) + k * LANES
               for k in range(EMBED // LANES)]
        gbufs = (gbuf0, gbuf1)
        blks = (blk0, blk1)

        for idx_hbm, tab, out_hbm in ((src_idx, src_tab, out_src),
                                      (tgt_idx, tgt_tab, out_tgt)):
            pltpu.sync_copy(idx_hbm.at[pl.ds(wid * idx_per_w, idx_per_w)],
                            idx_v)

            def repack_and_fire(g, gbuf, tab=tab):
                for c in range(2 * SBLK // LANES):
                    tt, sb = c // (SBLK // LANES), (c % (SBLK // LANES))
                    pos = ib50 + (sb * LANES * s_len + g * TT + tt)
                    vals = plsc.load_gather(idx_v, [pos])
                    list_v[pl.ds(c * LANES, LANES)] = vals
                for tt in range(TT):
                    pltpu.async_copy(
                        tab.at[list_v.at[pl.ds(tt * SBLK, SBLK)]],
                        gbuf.at[pl.ds(tt * SBLK, SBLK)], gsem)

            def drain_gathers(gbuf, tab=tab):
                pltpu.make_async_copy(tab.at[pl.ds(0, TT * SBLK)], gbuf,
                                      gsem).wait()

            def drain_stores(out_hbm=out_hbm):
                for blk in blks:
                    pltpu.make_async_copy(
                        blk.at[:, pl.ds(0, SBLK)],
                        out_hbm.at[0, :, pl.ds(wid * SBLK, SBLK)],
                        ssem).wait()

            repack_and_fire(0, gbufs[0])

            @pl.loop(0, n_groups + n_groups % 2, step=2)
            def _(g, tab=tab, out_hbm=out_hbm):
                for b in range(2):
                    gg = g + b

                    @pl.when(gg < n_groups)
                    def _():
                        drain_gathers(gbufs[b])

                        @pl.when(gg + 1 < n_groups)
                        def _():
                            repack_and_fire(gg + 1, gbufs[1 - b])

                        @pl.when(gg > 0)
                        def _():
                            drain_stores()

                        for tt in range(TT):
                            blk = blks[tt]

                            @pl.loop(0, SBLK // LANES)
                            def _(c, tt=tt, blk=blk):
                                for j in range(LANES):
                                    s = c * LANES + j
                                    col = jnp.full((LANES,), s, jnp.int32)
                                    for k in range(EMBED // LANES):
                                        v = gbufs[b][tt * SBLK + s,
                                                     pl.ds(k * LANES, LANES)]
                                        plsc.store_scatter(
                                            blk, [iks[k], col], v)

                            pltpu.async_copy(
                                blk.at[:, pl.ds(0, SBLK)],
                                out_hbm.at[gg * TT + tt, :,
                                           pl.ds(wid * SBLK, SBLK)], ssem)
            drain_stores()

    out_sd = jax.ShapeDtypeStruct((s_len, EMBED, n_sent), jnp.float32)
    return pl.kernel(
        body,
        out_type=(out_sd, out_sd),
        mesh=mesh,
        scratch_types=[
            pltpu.VMEM((SBLK * s_len,), jnp.int32),
            pltpu.VMEM((TT * SBLK,), jnp.int32),
            pltpu.VMEM((TT * SBLK, PADW), jnp.float32),
            pltpu.VMEM((TT * SBLK, PADW), jnp.float32),
            pltpu.VMEM((EMBED, SBLK + 1), jnp.float32),
            pltpu.VMEM((EMBED, SBLK + 1), jnp.float32),
            pltpu.SemaphoreType.DMA,
            pltpu.SemaphoreType.DMA,
        ],
        compiler_params=pltpu.CompilerParams(needs_layout_passes=False),
    )


def kernel(src_indices, tgt_indices, src_table, tgt_table):
    b, s = src_indices.shape
    src_flat = src_indices.reshape(b * s).astype(jnp.int32)
    tgt_flat = tgt_indices.reshape(b * s).astype(jnp.int32)
    src_pad = jnp.pad(src_table, ((0, 0), (0, PADW - EMBED)))
    tgt_pad = jnp.pad(tgt_table, ((0, 0), (0, PADW - EMBED)))
    p_src, p_tgt = _make_lookup(b, s)(src_flat, tgt_flat, src_pad, tgt_pad)
    return (jnp.transpose(p_src, (2, 0, 1)), jnp.transpose(p_tgt, (2, 0, 1)))


# speedup vs baseline: 1.4311x; 1.2156x over previous
import jax
import jax.numpy as jnp
from jax import lax
from jax.experimental import pallas as pl
from jax.experimental.pallas import tpu as pltpu
from jax.experimental.pallas import tpu_sc as plsc

EMBED = 64
PADW = 128
SBLK = 128
TT = 2
NC, NS = 2, 16
NW = NC * NS
LANES = 16


def _make_lookup(n_sent: int, s_len: int):
    assert n_sent == NW * SBLK and s_len % TT == 0
    n_groups = s_len // TT
    idx_per_w = SBLK * s_len
    mesh = plsc.VectorSubcoreMesh(core_axis_name="c", subcore_axis_name="s",
                                  num_cores=NC, num_subcores=NS)

    def body(src_idx, tgt_idx, src_tab, tgt_tab, out_src, out_tgt,
             idx_v, list_v, gbuf0, gbuf1, blk0, blk1, gsem, ssem):
        wid = lax.axis_index("s") * NC + lax.axis_index("c")
        ib50 = lax.iota(jnp.int32, LANES) * s_len
        iks = [lax.iota(jnp.int32, LANES) + k * LANES
               for k in range(EMBED // LANES)]
        gbufs = (gbuf0, gbuf1)
        blks = (blk0, blk1)

        for idx_hbm, tab, out_hbm in ((src_idx, src_tab, out_src),
                                      (tgt_idx, tgt_tab, out_tgt)):
            pltpu.sync_copy(idx_hbm.at[pl.ds(wid * idx_per_w, idx_per_w)],
                            idx_v)

            def repack_and_fire(g, gbuf, tab=tab):
                for c in range(2 * SBLK // LANES):
                    tt, sb = c // (SBLK // LANES), (c % (SBLK // LANES))
                    pos = ib50 + (sb * LANES * s_len + g * TT + tt)
                    vals = plsc.load_gather(idx_v, [pos])
                    list_v[pl.ds(c * LANES, LANES)] = vals
                for tt in range(TT):
                    pltpu.async_copy(
                        tab.at[list_v.at[pl.ds(tt * SBLK, SBLK)]],
                        gbuf.at[pl.ds(tt * SBLK, SBLK)], gsem)

            def drain_gathers(gbuf, tab=tab):
                pltpu.make_async_copy(tab.at[pl.ds(0, TT * SBLK)], gbuf,
                                      gsem).wait()

            def drain_stores(out_hbm=out_hbm):
                for blk in blks:
                    pltpu.make_async_copy(
                        blk.at[:, pl.ds(0, SBLK)],
                        out_hbm.at[0, :, pl.ds(wid * SBLK, SBLK)],
                        ssem).wait()

            repack_and_fire(0, gbufs[0])

            @pl.loop(0, n_groups + n_groups % 2, step=2)
            def _(g, tab=tab, out_hbm=out_hbm):
                for b in range(2):
                    gg = g + b

                    @pl.when(gg < n_groups)
                    def _():
                        drain_gathers(gbufs[b])

                        @pl.when(gg + 1 < n_groups)
                        def _():
                            repack_and_fire(gg + 1, gbufs[1 - b])

                        @pl.when(gg > 0)
                        def _():
                            drain_stores()

                        for tt in range(TT):
                            blk = blks[tt]

                            @plsc.parallel_loop(0, SBLK // LANES, unroll=2)
                            def _(c, tt=tt, blk=blk):
                                for j in range(LANES):
                                    s = c * LANES + j
                                    col = jnp.full((LANES,), s, jnp.int32)
                                    for k in range(EMBED // LANES):
                                        v = gbufs[b][tt * SBLK + s,
                                                     pl.ds(k * LANES, LANES)]
                                        plsc.store_scatter(
                                            blk, [iks[k], col], v)

                            pltpu.async_copy(
                                blk.at[:, pl.ds(0, SBLK)],
                                out_hbm.at[gg * TT + tt, :,
                                           pl.ds(wid * SBLK, SBLK)], ssem)
            drain_stores()

    out_sd = jax.ShapeDtypeStruct((s_len, EMBED, n_sent), jnp.float32)
    return pl.kernel(
        body,
        out_type=(out_sd, out_sd),
        mesh=mesh,
        scratch_types=[
            pltpu.VMEM((SBLK * s_len,), jnp.int32),
            pltpu.VMEM((TT * SBLK,), jnp.int32),
            pltpu.VMEM((TT * SBLK, PADW), jnp.float32),
            pltpu.VMEM((TT * SBLK, PADW), jnp.float32),
            pltpu.VMEM((EMBED, SBLK + 1), jnp.float32),
            pltpu.VMEM((EMBED, SBLK + 1), jnp.float32),
            pltpu.SemaphoreType.DMA,
            pltpu.SemaphoreType.DMA,
        ],
        compiler_params=pltpu.CompilerParams(needs_layout_passes=False),
    )


def kernel(src_indices, tgt_indices, src_table, tgt_table):
    b, s = src_indices.shape
    src_flat = src_indices.reshape(b * s).astype(jnp.int32)
    tgt_flat = tgt_indices.reshape(b * s).astype(jnp.int32)
    src_pad = jnp.pad(src_table, ((0, 0), (0, PADW - EMBED)))
    tgt_pad = jnp.pad(tgt_table, ((0, 0), (0, PADW - EMBED)))
    p_src, p_tgt = _make_lookup(b, s)(src_flat, tgt_flat, src_pad, tgt_pad)
    return (jnp.transpose(p_src, (2, 0, 1)), jnp.transpose(p_tgt, (2, 0, 1)))


# speedup vs baseline: 1.4827x; 1.0361x over previous
import jax
import jax.numpy as jnp
from jax import lax
from jax.experimental import pallas as pl
from jax.experimental.pallas import tpu as pltpu
from jax.experimental.pallas import tpu_sc as plsc

EMBED = 64
PADW = 128
SBLK = 128
TT = 2
NC, NS = 2, 16
NW = NC * NS
LANES = 16


def _make_lookup(n_sent: int, s_len: int):
    assert n_sent == NW * SBLK and s_len % TT == 0
    n_groups = s_len // TT
    idx_per_w = SBLK * s_len
    mesh = plsc.VectorSubcoreMesh(core_axis_name="c", subcore_axis_name="s",
                                  num_cores=NC, num_subcores=NS)

    def body(src_idx, tgt_idx, src_tab, tgt_tab, out_src, out_tgt,
             idx_v, list_v, gbuf0, gbuf1, blk0, blk1, gsem, ssem):
        wid = lax.axis_index("s") * NC + lax.axis_index("c")
        ib50 = lax.iota(jnp.int32, LANES) * s_len
        iks = [lax.iota(jnp.int32, LANES) + k * LANES
               for k in range(EMBED // LANES)]
        gbufs = (gbuf0, gbuf1)
        blks = (blk0, blk1)

        for idx_hbm, tab, out_hbm in ((src_idx, src_tab, out_src),
                                      (tgt_idx, tgt_tab, out_tgt)):
            pltpu.sync_copy(idx_hbm.at[pl.ds(wid * idx_per_w, idx_per_w)],
                            idx_v)

            def repack_and_fire(g, gbuf, tab=tab):
                for c in range(2 * SBLK // LANES):
                    tt, sb = c // (SBLK // LANES), (c % (SBLK // LANES))
                    pos = ib50 + (sb * LANES * s_len + g * TT + tt)
                    vals = plsc.load_gather(idx_v, [pos])
                    list_v[pl.ds(c * LANES, LANES)] = vals
                for tt in range(TT):
                    pltpu.async_copy(
                        tab.at[list_v.at[pl.ds(tt * SBLK, SBLK)]],
                        gbuf.at[pl.ds(tt * SBLK, SBLK)], gsem)

            def drain_gathers(gbuf, tab=tab):
                pltpu.make_async_copy(tab.at[pl.ds(0, TT * SBLK)], gbuf,
                                      gsem).wait()

            def drain_stores(out_hbm=out_hbm):
                for blk in blks:
                    pltpu.make_async_copy(
                        blk.at[:, pl.ds(0, SBLK)],
                        out_hbm.at[0, :, pl.ds(wid * SBLK, SBLK)],
                        ssem).wait()

            repack_and_fire(0, gbufs[0])

            @pl.loop(0, n_groups + n_groups % 2, step=2)
            def _(g, tab=tab, out_hbm=out_hbm):
                for b in range(2):
                    gg = g + b

                    @pl.when(gg < n_groups)
                    def _():
                        drain_gathers(gbufs[b])

                        @pl.when(gg + 1 < n_groups)
                        def _():
                            repack_and_fire(gg + 1, gbufs[1 - b])

                        @pl.when(gg > 0)
                        def _():
                            drain_stores()

                        for tt in range(TT):
                            blk = blks[tt]

                            @plsc.parallel_loop(0, SBLK // LANES, unroll=4)
                            def _(c, tt=tt, blk=blk):
                                for j in range(LANES):
                                    s = c * LANES + j
                                    col = jnp.full((LANES,), s, jnp.int32)
                                    for k in range(EMBED // LANES):
                                        v = gbufs[b][tt * SBLK + s,
                                                     pl.ds(k * LANES, LANES)]
                                        plsc.store_scatter(
                                            blk, [iks[k], col], v)

                            pltpu.async_copy(
                                blk.at[:, pl.ds(0, SBLK)],
                                out_hbm.at[gg * TT + tt, :,
                                           pl.ds(wid * SBLK, SBLK)], ssem)
            drain_stores()

    out_sd = jax.ShapeDtypeStruct((s_len, EMBED, n_sent), jnp.float32)
    return pl.kernel(
        body,
        out_type=(out_sd, out_sd),
        mesh=mesh,
        scratch_types=[
            pltpu.VMEM((SBLK * s_len,), jnp.int32),
            pltpu.VMEM((TT * SBLK,), jnp.int32),
            pltpu.VMEM((TT * SBLK, PADW), jnp.float32),
            pltpu.VMEM((TT * SBLK, PADW), jnp.float32),
            pltpu.VMEM((EMBED, SBLK + 1), jnp.float32),
            pltpu.VMEM((EMBED, SBLK + 1), jnp.float32),
            pltpu.SemaphoreType.DMA,
            pltpu.SemaphoreType.DMA,
        ],
        compiler_params=pltpu.CompilerParams(needs_layout_passes=False),
    )


def kernel(src_indices, tgt_indices, src_table, tgt_table):
    b, s = src_indices.shape
    src_flat = src_indices.reshape(b * s).astype(jnp.int32)
    tgt_flat = tgt_indices.reshape(b * s).astype(jnp.int32)
    src_pad = jnp.pad(src_table, ((0, 0), (0, PADW - EMBED)))
    tgt_pad = jnp.pad(tgt_table, ((0, 0), (0, PADW - EMBED)))
    p_src, p_tgt = _make_lookup(b, s)(src_flat, tgt_flat, src_pad, tgt_pad)
    return (jnp.transpose(p_src, (2, 0, 1)), jnp.transpose(p_tgt, (2, 0, 1)))


# speedup vs baseline: 1.8177x; 1.2259x over previous
import jax
import jax.numpy as jnp
from jax import lax
from jax.experimental import pallas as pl
from jax.experimental.pallas import tpu as pltpu
from jax.experimental.pallas import tpu_sc as plsc

EMBED = 64
GROUP = 8
NC, NS = 2, 16
NW = NC * NS


def _make_gather(n_sent: int, s_len: int):
    sent_per_w = n_sent // NW
    n_groups = sent_per_w // GROUP
    assert n_sent % NW == 0 and sent_per_w % GROUP == 0 and n_groups % 2 == 0
    mesh = plsc.VectorSubcoreMesh(core_axis_name="c", subcore_axis_name="s",
                                  num_cores=NC, num_subcores=NS)

    def body(idx_hbm, tab, out_hbm, idx_v, buf0, buf1, sem):
        wid = lax.axis_index("s") * NC + lax.axis_index("c")
        base = wid * sent_per_w
        bufs = (buf0, buf1)

        pltpu.sync_copy(idx_hbm.at[pl.ds(base, sent_per_w)], idx_v)

        def fire(g, buf):
            for j in range(GROUP):
                pltpu.async_copy(
                    tab.at[idx_v.at[g * GROUP + j]], buf.at[j], sem)

        def drain(buf):
            for j in range(GROUP):
                pltpu.make_async_copy(
                    tab.at[pl.ds(0, s_len)], buf.at[j], sem).wait()

        fire(0, bufs[0])

        @pl.loop(0, n_groups, step=2)
        def _(g):
            for b in range(2):
                gg = g + b
                drain(bufs[b])

                @pl.when(gg + 1 < n_groups)
                def _():
                    fire(gg + 1, bufs[1 - b])

                pltpu.sync_copy(
                    bufs[b], out_hbm.at[pl.ds(base + gg * GROUP, GROUP)])

    out_sd = jax.ShapeDtypeStruct((n_sent, s_len, EMBED), jnp.float32)
    return pl.kernel(
        body,
        out_type=out_sd,
        mesh=mesh,
        scratch_types=[
            pltpu.VMEM((sent_per_w, s_len), jnp.int32),
            pltpu.VMEM((GROUP, s_len, EMBED), jnp.float32),
            pltpu.VMEM((GROUP, s_len, EMBED), jnp.float32),
            pltpu.SemaphoreType.DMA,
        ],
        compiler_params=pltpu.CompilerParams(use_tc_tiling_on_sc=False),
    )


def kernel(src_indices, tgt_indices, src_table, tgt_table):
    b, s = src_indices.shape
    lookup = _make_gather(b, s)
    out_src = lookup(src_indices.astype(jnp.int32), src_table)
    out_tgt = lookup(tgt_indices.astype(jnp.int32), tgt_table)
    return (out_src, out_tgt)
